# baseline (speedup 1.0000x reference)
import functools

import jax
import jax.numpy as jnp
from jax import lax
from jax.experimental import pallas as pl
from jax.experimental.pallas import tpu as pltpu
from jax.experimental.pallas import tpu_sc as plsc

F32 = jnp.float32
BF16 = jnp.bfloat16
NC = 2
NS = 16
LANES = 16


def _pack_pair(lo_f32, hi_f32):
    lo = lax.bitcast_convert_type(
        lo_f32.astype(BF16), jnp.uint16).astype(jnp.uint32)
    hi = lax.bitcast_convert_type(
        hi_f32.astype(BF16), jnp.uint16).astype(jnp.uint32)
    return lax.bitcast_convert_type(lo | (hi << jnp.uint32(16)), jnp.int32)


def _proj_body(x_ref, w_ref, p_ref):
    x = x_ref[...].astype(BF16)
    d = x.shape[1]
    w = w_ref[...].astype(BF16)
    a = jnp.dot(x, w[0:d, :], preferred_element_type=F32)
    b = jnp.dot(x, w[d:2 * d, :], preferred_element_type=F32)
    p_ref[...] = _pack_pair(a, b)


def _proj(X, edge_W, blk):
    N, D = X.shape
    DH = D // NC
    nb = N // blk
    return pl.pallas_call(
        _proj_body,
        grid=(nb, NC),
        in_specs=[
            pl.BlockSpec((blk, D), lambda i, j: (i, 0)),
            pl.BlockSpec((3 * D, DH), lambda i, j: (0, j)),
        ],
        out_specs=pl.BlockSpec((blk, DH), lambda i, j: (j * nb + i, 0)),
        out_shape=jax.ShapeDtypeStruct((NC * N, DH), jnp.int32),
    )(X, edge_W)


def _ce_body(e_ref, w3_ref, b_ref, ce_ref):
    e = e_ref[...]
    c = (jnp.dot(e.astype(BF16), w3_ref[...].astype(BF16),
                 preferred_element_type=F32) + b_ref[...])
    ce_ref[...] = _pack_pair(c, e)


def _ce_pack(E, W3, b2d, blk):
    En, D = E.shape
    blk_spec = pl.BlockSpec((blk, D), lambda i: (i, 0))
    return pl.pallas_call(
        _ce_body,
        grid=(En // blk,),
        in_specs=[
            blk_spec,
            pl.BlockSpec((D, D), lambda i: (0, 0)),
            pl.BlockSpec((1, D), lambda i: (0, 0)),
        ],
        out_specs=blk_spec,
        out_shape=jax.ShapeDtypeStruct((En, D), jnp.int32),
    )(E, W3, b2d)


def _node_body(ag_ref, x_ref, w_ref, b_ref, out_ref):
    x = x_ref[...]
    d = x.shape[1]
    h = (jnp.dot(ag_ref[...], w_ref[0:d, :], preferred_element_type=F32)
         + jnp.dot(x, w_ref[d:2 * d, :], preferred_element_type=F32)
         + b_ref[...])
    out_ref[...] = jnp.where(h > 0, h, jnp.exp(h) - 1.0) + x


def _node_mlp(aggr, X, node_W, b2d, blk):
    N, D = X.shape
    blk_spec = pl.BlockSpec((blk, D), lambda i: (i, 0))
    return pl.pallas_call(
        _node_body,
        grid=(N // blk,),
        in_specs=[
            blk_spec, blk_spec,
            pl.BlockSpec((2 * D, D), lambda i: (0, 0)),
            pl.BlockSpec((1, D), lambda i: (0, 0)),
        ],
        out_specs=blk_spec,
        out_shape=jax.ShapeDtypeStruct((N, D), F32),
    )(aggr, X, node_W, b2d)


def _sc_mega(P2, CE, row, col, N, CB):
    En, D = CE.shape
    DH = D // NC
    ept = En // NS
    assert En % NS == 0 and ept % CB == 0
    assert CB % 8 == 0
    nchunks = ept // CB
    RB = 40
    assert N % RB == 0
    nrb = N // RB
    nsl = DH // LANES
    ioffs = list(range(0, CB - LANES + 1, LANES))
    if CB % LANES:
        ioffs.append(CB - LANES)
    mesh = plsc.VectorSubcoreMesh(core_axis_name="c", subcore_axis_name="s")

    @functools.partial(
        pl.kernel,
        out_type=[
            jax.ShapeDtypeStruct((En, D), F32),
            jax.ShapeDtypeStruct((N, D), F32),
        ],
        mesh=mesh,
        scratch_types=[
            pltpu.VMEM_SHARED((N, DH), F32),
            pltpu.VMEM((2, CB, DH), jnp.int32),
            pltpu.VMEM((2, CB, DH), jnp.int32),
            pltpu.VMEM((2, CB, DH), jnp.int32),
            pltpu.VMEM((2, CB, DH), F32),
            pltpu.VMEM((CB, DH), F32),
            pltpu.VMEM((2, CB), jnp.int32),
            pltpu.VMEM((5, CB), jnp.int32),
            pltpu.VMEM((2, CB), jnp.int32),
            pltpu.VMEM((2, CB), jnp.int32),
            pltpu.SemaphoreType.DMA,
            pltpu.SemaphoreType.DMA,
            pltpu.SemaphoreType.DMA,
            pltpu.SemaphoreType.DMA,
            pltpu.SemaphoreType.DMA,
            pltpu.SemaphoreType.DMA,
            pltpu.SemaphoreType.DMA,
            pltpu.SemaphoreType.DMA,
        ],
    )
    def mega_kernel(p2_hbm, ce_hbm, row_hbm, col_hbm, enew_hbm, aggr_hbm,
                    acc, bPc, bPr, bCE, bMsg, bEn, ixr, ixc, ixr2,
                    ixc2, semg0, semg1, semi0, semi1, semw0, semw1,
                    semsc0, semsc1):
        c = lax.axis_index("c")
        s = lax.axis_index("s")
        semg = (semg0, semg1)
        semi = (semi0, semi1)
        semw = (semw0, semw1)
        semsc = (semsc0, semsc1)
        coff = c * DH
        cN = c * N

        zb = bMsg.at[0]

        def zrow(i, carry):
            for j in range(nsl):
                bMsg[0, i, pl.ds(j * LANES, LANES)] = jnp.zeros((LANES,),
                                                                F32)
            return carry

        lax.fori_loop(0, RB, zrow, 0)
        nblk = (nrb - s + NS - 1) // NS

        def zblk(k, carry):
            roff = (s + k * NS) * RB
            pltpu.sync_copy(zb, acc.at[pl.ds(roff, RB)])
            return carry

        lax.fori_loop(0, nblk, zblk, 0)
        plsc.subcore_barrier()

        def fire_idx(t, S):
            m = lax.rem(t, 5)
            eoff = s * ept + t * CB
            pltpu.async_copy(row_hbm.at[pl.ds(eoff, CB)], ixr.at[S],
                             semi[S])
            pltpu.async_copy(col_hbm.at[pl.ds(eoff, CB)], ixc.at[m],
                             semi[S])

        def wait_scatter(S):
            pltpu.make_async_copy(bMsg.at[S], acc.at[ixc.at[0]],
                                  semsc[S]).wait()

        def fire_gather(t, S):
            m = lax.rem(t, 5)
            eoff = s * ept + t * CB
            pltpu.make_async_copy(row_hbm.at[pl.ds(0, CB)], ixr.at[S],
                                  semi[S]).wait()
            pltpu.make_async_copy(col_hbm.at[pl.ds(0, CB)], ixc.at[0],
                                  semi[S]).wait()
            for o in ioffs:
                sl = pl.ds(o, LANES)
                ixr2[S, sl] = ixr[S, sl] + cN
                ixc2[S, sl] = ixc[m, sl] + cN
            pltpu.async_copy(p2_hbm.at[ixc2.at[S]], bPc.at[S], semg[S])
            pltpu.async_copy(p2_hbm.at[ixr2.at[S]], bPr.at[S], semg[S])
            pltpu.async_copy(ce_hbm.at[pl.ds(eoff, CB), pl.ds(coff, DH)],
                             bCE.at[S], semg[S])

        def finish(t, S, first):
            if not first:
                wait_scatter(S)
            pltpu.make_async_copy(p2_hbm.at[ixc2.at[S]], bPc.at[S],
                                  semg[S]).wait()
            pltpu.make_async_copy(p2_hbm.at[ixr2.at[S]], bPr.at[S],
                                  semg[S]).wait()
            pltpu.make_async_copy(
                ce_hbm.at[pl.ds(0, CB), pl.ds(0, DH)],
                bCE.at[S], semg[S]).wait()

            bc_ = lax.bitcast_convert_type
            sh = jnp.uint32(16)
            hi = jnp.uint32(0xFFFF0000)

            def erow(i, carry):
                for j in range(nsl):
                    sl = pl.ds(j * LANES, LANES)
                    cv = bc_(bPc[S, i, sl], jnp.uint32)
                    rv = bc_(bPr[S, i, sl], jnp.uint32)
                    ce = bc_(bCE[S, i, sl], jnp.uint32)
                    ac = bc_(cv << sh, F32)
                    bcv = bc_(cv & hi, F32)
                    ar = bc_(rv << sh, F32)
                    br = bc_(rv & hi, F32)
                    cf = bc_(ce << sh, F32)
                    ef = bc_(ce & hi, F32)
                    pm = ac + br + cf
                    pn = ar + bcv + cf
                    bMsg[S, i, sl] = jnp.where(pm > 0, pm,
                                               jnp.exp(pm) - 1.0)
                    bEn[i, sl] = jnp.where(pn > 0, pn,
                                           jnp.exp(pn) - 1.0) + ef
                return carry

            lax.fori_loop(0, CB, erow, 0, unroll=2)
            m = lax.rem(t, 5)
            eoff = s * ept + t * CB
            pltpu.async_copy(
                bEn,
                enew_hbm.at[pl.ds(eoff, CB), pl.ds(coff, DH)], semw[S])
            pltpu.async_copy(bMsg.at[S], acc.at[ixc.at[m]], semsc[S],
                             add=True)
            pltpu.make_async_copy(
                bEn,
                enew_hbm.at[pl.ds(0, CB), pl.ds(0, DH)], semw[S]).wait()

        assert nchunks % 2 == 0 and nchunks >= 6
        fire_idx(0, 0)
        fire_idx(1, 1)
        fire_gather(0, 0)
        fire_gather(1, 1)
        fire_idx(2, 0)
        finish(0, 0, first=True)
        fire_gather(2, 0)
        fire_idx(3, 1)
        finish(1, 1, first=True)

        def pair_body(p, carry):
            t = 2 + 2 * p
            fire_gather(t + 1, 1)
            fire_idx(t + 2, 0)
            finish(t, 0, first=False)
            fire_gather(t + 2, 0)
            fire_idx(t + 3, 1)
            finish(t + 1, 1, first=False)
            return carry

        lax.fori_loop(0, (nchunks - 4) // 2, pair_body, 0)
        fire_gather(nchunks - 1, 1)
        finish(nchunks - 2, 0, first=False)
        finish(nchunks - 1, 1, first=False)
        wait_scatter(0)
        wait_scatter(1)

        plsc.subcore_barrier()

        def wblk(k, carry):
            roff = (s + k * NS) * RB
            pltpu.sync_copy(acc.at[pl.ds(roff, RB)], zb)
            pltpu.sync_copy(zb, aggr_hbm.at[pl.ds(roff, RB),
                                            pl.ds(coff, DH)])
            return carry

        lax.fori_loop(0, nblk, wblk, 0)

    return mega_kernel(P2, CE, row, col)


def kernel(X, E, emb_nodes, emb_edges, edge_index, edge_W, edge_b,
           node_W, node_b):
    N, D = X.shape
    En = E.shape[0]
    row = edge_index[0]
    col = edge_index[1]
    eb2 = edge_b.reshape(1, D)
    nb2 = node_b.reshape(1, D)
    W3 = lax.slice(edge_W, (2 * D, 0), (3 * D, D))

    P2 = _proj(X, edge_W, blk=1000)
    CE = _ce_pack(E, W3, eb2, blk=2000)
    E_new, aggr = _sc_mega(P2, CE, row, col, N, CB=40)
    X_new = _node_mlp(aggr, X, node_W, nb2, blk=1000)
    return X_new, E_new


# speedup vs baseline: 2.7103x; 2.7103x over previous
import functools

import jax
import jax.numpy as jnp
from jax import lax
from jax.experimental import pallas as pl
from jax.experimental.pallas import tpu as pltpu
from jax.experimental.pallas import tpu_sc as plsc

F32 = jnp.float32
BF16 = jnp.bfloat16
NC = 2
NS = 16
LANES = 16


def _pack_pair(lo_f32, hi_f32):
    lo = lax.bitcast_convert_type(
        lo_f32.astype(BF16), jnp.uint16).astype(jnp.uint32)
    hi = lax.bitcast_convert_type(
        hi_f32.astype(BF16), jnp.uint16).astype(jnp.uint32)
    return lax.bitcast_convert_type(lo | (hi << jnp.uint32(16)), jnp.int32)


def _proj_body(x_ref, w_ref, p_ref):
    x = x_ref[...].astype(BF16)
    d = x.shape[1]
    w = w_ref[...].astype(BF16)
    a = jnp.dot(x, w[0:d, :], preferred_element_type=F32)
    b = jnp.dot(x, w[d:2 * d, :], preferred_element_type=F32)
    p_ref[...] = _pack_pair(a, b)


def _proj(X, edge_W, blk):
    N, D = X.shape
    DH = D // NC
    nb = N // blk
    return pl.pallas_call(
        _proj_body,
        grid=(nb, NC),
        in_specs=[
            pl.BlockSpec((blk, D), lambda i, j: (i, 0)),
            pl.BlockSpec((3 * D, DH), lambda i, j: (0, j)),
        ],
        out_specs=pl.BlockSpec((blk, DH), lambda i, j: (j * nb + i, 0)),
        out_shape=jax.ShapeDtypeStruct((NC * N, DH), jnp.int32),
    )(X, edge_W)


def _ce_body(e_ref, w3_ref, b_ref, ce_ref):
    e = e_ref[...]
    c = (jnp.dot(e.astype(BF16), w3_ref[...].astype(BF16),
                 preferred_element_type=F32) + b_ref[...])
    ce_ref[...] = _pack_pair(c, e)


def _ce_pack(E, W3, b2d, blk):
    En, D = E.shape
    blk_spec = pl.BlockSpec((blk, D), lambda i: (i, 0))
    return pl.pallas_call(
        _ce_body,
        grid=(En // blk,),
        in_specs=[
            blk_spec,
            pl.BlockSpec((D, D), lambda i: (0, 0)),
            pl.BlockSpec((1, D), lambda i: (0, 0)),
        ],
        out_specs=blk_spec,
        out_shape=jax.ShapeDtypeStruct((En, D), jnp.int32),
    )(E, W3, b2d)


def _node_body(ag_ref, x_ref, w_ref, b_ref, out_ref):
    x = x_ref[...]
    d = x.shape[1]
    h = (jnp.dot(ag_ref[...], w_ref[0:d, :], preferred_element_type=F32)
         + jnp.dot(x, w_ref[d:2 * d, :], preferred_element_type=F32)
         + b_ref[...])
    out_ref[...] = jnp.where(h > 0, h, jnp.exp(h) - 1.0) + x


def _node_mlp(aggr, X, node_W, b2d, blk):
    N, D = X.shape
    blk_spec = pl.BlockSpec((blk, D), lambda i: (i, 0))
    return pl.pallas_call(
        _node_body,
        grid=(N // blk,),
        in_specs=[
            blk_spec, blk_spec,
            pl.BlockSpec((2 * D, D), lambda i: (0, 0)),
            pl.BlockSpec((1, D), lambda i: (0, 0)),
        ],
        out_specs=blk_spec,
        out_shape=jax.ShapeDtypeStruct((N, D), F32),
    )(aggr, X, node_W, b2d)


def _sc_mega(P2, CE, row, col, N, CB):
    En, D = CE.shape
    DH = D // NC
    ept = En // NS
    assert En % NS == 0 and ept % CB == 0
    assert CB % 8 == 0
    nchunks = ept // CB
    RB = 40
    assert N % RB == 0
    nrb = N // RB
    nsl = DH // LANES
    ioffs = list(range(0, CB - LANES + 1, LANES))
    if CB % LANES:
        ioffs.append(CB - LANES)
    mesh = plsc.VectorSubcoreMesh(core_axis_name="c", subcore_axis_name="s")

    @functools.partial(
        pl.kernel,
        out_type=[
            jax.ShapeDtypeStruct((En, D), F32),
            jax.ShapeDtypeStruct((N, D), F32),
        ],
        mesh=mesh,
        scratch_types=[
            pltpu.VMEM_SHARED((N, DH), F32),
            pltpu.VMEM((2, CB, DH), jnp.int32),
            pltpu.VMEM((2, CB, DH), jnp.int32),
            pltpu.VMEM((2, CB, DH), jnp.int32),
            pltpu.VMEM((2, CB, DH), F32),
            pltpu.VMEM((CB, DH), F32),
            pltpu.VMEM((2, CB), jnp.int32),
            pltpu.VMEM((5, CB), jnp.int32),
            pltpu.VMEM((2, CB), jnp.int32),
            pltpu.VMEM((2, CB), jnp.int32),
            pltpu.SemaphoreType.DMA,
            pltpu.SemaphoreType.DMA,
            pltpu.SemaphoreType.DMA,
            pltpu.SemaphoreType.DMA,
            pltpu.SemaphoreType.DMA,
            pltpu.SemaphoreType.DMA,
            pltpu.SemaphoreType.DMA,
            pltpu.SemaphoreType.DMA,
        ],
    )
    def mega_kernel(p2_hbm, ce_hbm, row_hbm, col_hbm, enew_hbm, aggr_hbm,
                    acc, bPc, bPr, bCE, bMsg, bEn, ixr, ixc, ixr2,
                    ixc2, semg0, semg1, semi0, semi1, semw0, semw1,
                    semsc0, semsc1):
        c = lax.axis_index("c")
        s = lax.axis_index("s")
        semg = (semg0, semg1)
        semi = (semi0, semi1)
        semw = (semw0, semw1)
        semsc = (semsc0, semsc1)
        coff = c * DH
        cN = c * N

        zb = bMsg.at[0]

        def zrow(i, carry):
            for j in range(nsl):
                bMsg[0, i, pl.ds(j * LANES, LANES)] = jnp.zeros((LANES,),
                                                                F32)
            return carry

        lax.fori_loop(0, RB, zrow, 0)
        nblk = (nrb - s + NS - 1) // NS

        def zblk(k, carry):
            roff = (s + k * NS) * RB
            pltpu.sync_copy(zb, acc.at[pl.ds(roff, RB)])
            return carry

        lax.fori_loop(0, nblk, zblk, 0)
        plsc.subcore_barrier()

        def fire_idx(t, S):
            m = lax.rem(t, 5)
            eoff = s * ept + t * CB
            pltpu.async_copy(row_hbm.at[pl.ds(eoff, CB)], ixr.at[S],
                             semi[S])
            pltpu.async_copy(col_hbm.at[pl.ds(eoff, CB)], ixc.at[m],
                             semi[S])

        def wait_scatter(S):
            pltpu.make_async_copy(bMsg.at[S], acc.at[ixc.at[0]],
                                  semsc[S]).wait()

        def fire_gather(t, S):
            m = lax.rem(t, 5)
            eoff = s * ept + t * CB
            pltpu.make_async_copy(row_hbm.at[pl.ds(0, CB)], ixr.at[S],
                                  semi[S]).wait()
            pltpu.make_async_copy(col_hbm.at[pl.ds(0, CB)], ixc.at[0],
                                  semi[S]).wait()
            for o in ioffs:
                sl = pl.ds(o, LANES)
                ixr2[S, sl] = ixr[S, sl] + cN
                ixc2[S, sl] = ixc[m, sl] + cN
            pltpu.async_copy(p2_hbm.at[ixc2.at[S]], bPc.at[S], semg[S])
            pltpu.async_copy(p2_hbm.at[ixr2.at[S]], bPr.at[S], semg[S])
            pltpu.async_copy(ce_hbm.at[pl.ds(eoff, CB), pl.ds(coff, DH)],
                             bCE.at[S], semg[S])

        def finish(t, S, first):
            if not first:
                wait_scatter(S)
            pltpu.make_async_copy(p2_hbm.at[ixc2.at[S]], bPc.at[S],
                                  semg[S]).wait()
            pltpu.make_async_copy(p2_hbm.at[ixr2.at[S]], bPr.at[S],
                                  semg[S]).wait()
            pltpu.make_async_copy(
                ce_hbm.at[pl.ds(0, CB), pl.ds(0, DH)],
                bCE.at[S], semg[S]).wait()

            bc_ = lax.bitcast_convert_type
            sh = jnp.uint32(16)
            hi = jnp.uint32(0xFFFF0000)

            def erow(i, carry):
                for j in range(nsl):
                    sl = pl.ds(j * LANES, LANES)
                    cv = bc_(bPc[S, i, sl], jnp.uint32)
                    rv = bc_(bPr[S, i, sl], jnp.uint32)
                    ce = bc_(bCE[S, i, sl], jnp.uint32)
                    ac = bc_(cv << sh, F32)
                    bcv = bc_(cv & hi, F32)
                    ar = bc_(rv << sh, F32)
                    br = bc_(rv & hi, F32)
                    cf = bc_(ce << sh, F32)
                    ef = bc_(ce & hi, F32)
                    pm = ac + br + cf
                    pn = ar + bcv + cf
                    bMsg[S, i, sl] = jnp.where(pm > 0, pm,
                                               jnp.exp(pm) - 1.0)
                    bEn[i, sl] = jnp.where(pn > 0, pn,
                                           jnp.exp(pn) - 1.0) + ef
                return carry

            lax.fori_loop(0, CB, erow, 0)
            m = lax.rem(t, 5)
            eoff = s * ept + t * CB
            pltpu.async_copy(
                bEn,
                enew_hbm.at[pl.ds(eoff, CB), pl.ds(coff, DH)], semw[S])
            pltpu.async_copy(bMsg.at[S], acc.at[ixc.at[m]], semsc[S],
                             add=True)
            pltpu.make_async_copy(
                bEn,
                enew_hbm.at[pl.ds(0, CB), pl.ds(0, DH)], semw[S]).wait()

        assert nchunks % 2 == 0 and nchunks >= 6
        fire_idx(0, 0)
        fire_idx(1, 1)
        fire_gather(0, 0)
        fire_gather(1, 1)
        fire_idx(2, 0)
        finish(0, 0, first=True)
        fire_gather(2, 0)
        fire_idx(3, 1)
        finish(1, 1, first=True)

        def pair_body(p, carry):
            t = 2 + 2 * p
            fire_gather(t + 1, 1)
            fire_idx(t + 2, 0)
            finish(t, 0, first=False)
            fire_gather(t + 2, 0)
            fire_idx(t + 3, 1)
            finish(t + 1, 1, first=False)
            return carry

        lax.fori_loop(0, (nchunks - 4) // 2, pair_body, 0)
        fire_gather(nchunks - 1, 1)
        finish(nchunks - 2, 0, first=False)
        finish(nchunks - 1, 1, first=False)
        wait_scatter(0)
        wait_scatter(1)

        plsc.subcore_barrier()

        def wblk(k, carry):
            roff = (s + k * NS) * RB
            pltpu.sync_copy(acc.at[pl.ds(roff, RB)], zb)
            pltpu.sync_copy(zb, aggr_hbm.at[pl.ds(roff, RB),
                                            pl.ds(coff, DH)])
            return carry

        lax.fori_loop(0, nblk, wblk, 0)

    return mega_kernel(P2, CE, row, col)


def kernel(X, E, emb_nodes, emb_edges, edge_index, edge_W, edge_b,
           node_W, node_b):
    N, D = X.shape
    En = E.shape[0]
    row = edge_index[0]
    col = edge_index[1]
    eb2 = edge_b.reshape(1, D)
    nb2 = node_b.reshape(1, D)
    W3 = lax.slice(edge_W, (2 * D, 0), (3 * D, D))

    P2 = _proj(X, edge_W, blk=1000)
    CE = _ce_pack(E, W3, eb2, blk=2000)
    E_new, aggr = _sc_mega(P2, CE, row, col, N, CB=40)
    X_new = _node_mlp(aggr, X, node_W, nb2, blk=1000)
    return X_new, E_new


# speedup vs baseline: 2.7500x; 1.0147x over previous
import functools

import jax
import jax.numpy as jnp
from jax import lax
from jax.experimental import pallas as pl
from jax.experimental.pallas import tpu as pltpu
from jax.experimental.pallas import tpu_sc as plsc

F32 = jnp.float32
BF16 = jnp.bfloat16
NC = 2
NS = 16
LANES = 16


def _pack_pair(lo_f32, hi_f32):
    lo = lax.bitcast_convert_type(
        lo_f32.astype(BF16), jnp.uint16).astype(jnp.uint32)
    hi = lax.bitcast_convert_type(
        hi_f32.astype(BF16), jnp.uint16).astype(jnp.uint32)
    return lax.bitcast_convert_type(lo | (hi << jnp.uint32(16)), jnp.int32)


def _proj_body(x_ref, w_ref, p_ref):
    x = x_ref[...].astype(BF16)
    d = x.shape[1]
    w = w_ref[...].astype(BF16)
    a = jnp.dot(x, w[0:d, :], preferred_element_type=F32)
    b = jnp.dot(x, w[d:2 * d, :], preferred_element_type=F32)
    p_ref[...] = _pack_pair(a, b)


def _proj(X, edge_W, blk):
    N, D = X.shape
    DH = D // NC
    nb = N // blk
    return pl.pallas_call(
        _proj_body,
        grid=(nb, NC),
        in_specs=[
            pl.BlockSpec((blk, D), lambda i, j: (i, 0)),
            pl.BlockSpec((3 * D, DH), lambda i, j: (0, j)),
        ],
        out_specs=pl.BlockSpec((blk, DH), lambda i, j: (j * nb + i, 0)),
        out_shape=jax.ShapeDtypeStruct((NC * N, DH), jnp.int32),
    )(X, edge_W)


def _ce_body(e_ref, w3_ref, b_ref, ce_ref):
    e = e_ref[...]
    c = (jnp.dot(e.astype(BF16), w3_ref[...].astype(BF16),
                 preferred_element_type=F32) + b_ref[...])
    ce_ref[...] = _pack_pair(c, e)


def _ce_pack(E, W3, b2d, blk):
    En, D = E.shape
    blk_spec = pl.BlockSpec((blk, D), lambda i: (i, 0))
    return pl.pallas_call(
        _ce_body,
        grid=(En // blk,),
        in_specs=[
            blk_spec,
            pl.BlockSpec((D, D), lambda i: (0, 0)),
            pl.BlockSpec((1, D), lambda i: (0, 0)),
        ],
        out_specs=blk_spec,
        out_shape=jax.ShapeDtypeStruct((En, D), jnp.int32),
    )(E, W3, b2d)


def _node_body(ag_ref, x_ref, w_ref, b_ref, out_ref):
    x = x_ref[...]
    d = x.shape[1]
    h = (jnp.dot(ag_ref[...], w_ref[0:d, :], preferred_element_type=F32)
         + jnp.dot(x, w_ref[d:2 * d, :], preferred_element_type=F32)
         + b_ref[...])
    out_ref[...] = jnp.where(h > 0, h, jnp.exp(h) - 1.0) + x


def _node_mlp(aggr, X, node_W, b2d, blk):
    N, D = X.shape
    blk_spec = pl.BlockSpec((blk, D), lambda i: (i, 0))
    return pl.pallas_call(
        _node_body,
        grid=(N // blk,),
        in_specs=[
            blk_spec, blk_spec,
            pl.BlockSpec((2 * D, D), lambda i: (0, 0)),
            pl.BlockSpec((1, D), lambda i: (0, 0)),
        ],
        out_specs=blk_spec,
        out_shape=jax.ShapeDtypeStruct((N, D), F32),
    )(aggr, X, node_W, b2d)


def _sc_mega(P2, CE, row, col, N, CB):
    En, D = CE.shape
    DH = D // NC
    ept = En // NS
    assert En % NS == 0 and ept % CB == 0
    assert CB % 8 == 0
    nchunks = ept // CB
    RB = 40
    assert N % RB == 0
    nrb = N // RB
    nsl = DH // LANES
    ioffs = list(range(0, CB - LANES + 1, LANES))
    if CB % LANES:
        ioffs.append(CB - LANES)
    mesh = plsc.VectorSubcoreMesh(core_axis_name="c", subcore_axis_name="s")

    @functools.partial(
        pl.kernel,
        out_type=[
            jax.ShapeDtypeStruct((En, D), F32),
            jax.ShapeDtypeStruct((N, D), F32),
        ],
        mesh=mesh,
        scratch_types=[
            pltpu.VMEM_SHARED((N, DH), F32),
            pltpu.VMEM((2, CB, DH), jnp.int32),
            pltpu.VMEM((2, CB, DH), jnp.int32),
            pltpu.VMEM((2, CB, DH), jnp.int32),
            pltpu.VMEM((2, CB, DH), F32),
            pltpu.VMEM((CB, DH), F32),
            pltpu.VMEM((2, CB), jnp.int32),
            pltpu.VMEM((5, CB), jnp.int32),
            pltpu.VMEM((2, CB), jnp.int32),
            pltpu.VMEM((2, CB), jnp.int32),
            pltpu.SemaphoreType.DMA,
            pltpu.SemaphoreType.DMA,
            pltpu.SemaphoreType.DMA,
            pltpu.SemaphoreType.DMA,
            pltpu.SemaphoreType.DMA,
            pltpu.SemaphoreType.DMA,
            pltpu.SemaphoreType.DMA,
            pltpu.SemaphoreType.DMA,
        ],
    )
    def mega_kernel(p2_hbm, ce_hbm, row_hbm, col_hbm, enew_hbm, aggr_hbm,
                    acc, bPc, bPr, bCE, bMsg, bEn, ixr, ixc, ixr2,
                    ixc2, semg0, semg1, semi0, semi1, semw0, semw1,
                    semsc0, semsc1):
        c = lax.axis_index("c")
        s = lax.axis_index("s")
        semg = (semg0, semg1)
        semi = (semi0, semi1)
        semw = (semw0, semw1)
        semsc = (semsc0, semsc1)
        coff = c * DH
        cN = c * N

        zb = bMsg.at[0]

        def zrow(i, carry):
            for j in range(nsl):
                bMsg[0, i, pl.ds(j * LANES, LANES)] = jnp.zeros((LANES,),
                                                                F32)
            return carry

        lax.fori_loop(0, RB, zrow, 0)
        nblk = (nrb - s + NS - 1) // NS

        def zblk(k, carry):
            roff = (s + k * NS) * RB
            pltpu.sync_copy(zb, acc.at[pl.ds(roff, RB)])
            return carry

        lax.fori_loop(0, nblk, zblk, 0)
        plsc.subcore_barrier()

        def fire_idx(t, S):
            m = lax.rem(t, 5)
            eoff = s * ept + t * CB
            pltpu.async_copy(row_hbm.at[pl.ds(eoff, CB)], ixr.at[S],
                             semi[S])
            pltpu.async_copy(col_hbm.at[pl.ds(eoff, CB)], ixc.at[m],
                             semi[S])

        def wait_scatter(S):
            pltpu.make_async_copy(bMsg.at[S], acc.at[ixc.at[0]],
                                  semsc[S]).wait()

        def fire_gather(t, S):
            m = lax.rem(t, 5)
            eoff = s * ept + t * CB
            pltpu.make_async_copy(row_hbm.at[pl.ds(0, CB)], ixr.at[S],
                                  semi[S]).wait()
            pltpu.make_async_copy(col_hbm.at[pl.ds(0, CB)], ixc.at[0],
                                  semi[S]).wait()
            for o in ioffs:
                sl = pl.ds(o, LANES)
                ixr2[S, sl] = ixr[S, sl] + cN
                ixc2[S, sl] = ixc[m, sl] + cN
            pltpu.async_copy(p2_hbm.at[ixc2.at[S]], bPc.at[S], semg[S])
            pltpu.async_copy(p2_hbm.at[ixr2.at[S]], bPr.at[S], semg[S])
            pltpu.async_copy(ce_hbm.at[pl.ds(eoff, CB), pl.ds(coff, DH)],
                             bCE.at[S], semg[S])

        def finish(t, S, first):
            if not first:
                wait_scatter(S)
            pltpu.make_async_copy(p2_hbm.at[ixc2.at[S]], bPc.at[S],
                                  semg[S]).wait()
            pltpu.make_async_copy(p2_hbm.at[ixr2.at[S]], bPr.at[S],
                                  semg[S]).wait()
            pltpu.make_async_copy(
                ce_hbm.at[pl.ds(0, CB), pl.ds(0, DH)],
                bCE.at[S], semg[S]).wait()

            bc_ = lax.bitcast_convert_type
            sh = jnp.uint32(16)
            hi = jnp.uint32(0xFFFF0000)

            def erow(i, carry):
                for j in range(nsl):
                    sl = pl.ds(j * LANES, LANES)
                    cv = bc_(bPc[S, i, sl], jnp.uint32)
                    rv = bc_(bPr[S, i, sl], jnp.uint32)
                    ce = bc_(bCE[S, i, sl], jnp.uint32)
                    ac = bc_(cv << sh, F32)
                    bcv = bc_(cv & hi, F32)
                    ar = bc_(rv << sh, F32)
                    br = bc_(rv & hi, F32)
                    cf = bc_(ce << sh, F32)
                    ef = bc_(ce & hi, F32)
                    pm = ac + br + cf
                    pn = ar + bcv + cf
                    bMsg[S, i, sl] = jnp.where(pm > 0, pm,
                                               jnp.exp(pm) - 1.0)
                    bEn[i, sl] = jnp.where(pn > 0, pn,
                                           jnp.exp(pn) - 1.0) + ef
                return carry

            lax.fori_loop(0, CB, erow, 0)
            m = lax.rem(t, 5)
            eoff = s * ept + t * CB
            pltpu.async_copy(
                bEn,
                enew_hbm.at[pl.ds(eoff, CB), pl.ds(coff, DH)], semw[S])
            pltpu.async_copy(bMsg.at[S], acc.at[ixc.at[m]], semsc[S],
                             add=True)
            pltpu.make_async_copy(
                bEn,
                enew_hbm.at[pl.ds(0, CB), pl.ds(0, DH)], semw[S]).wait()

        assert nchunks % 2 == 0 and nchunks >= 6
        fire_idx(0, 0)
        fire_idx(1, 1)
        fire_gather(0, 0)
        fire_gather(1, 1)
        fire_idx(2, 0)
        finish(0, 0, first=True)
        fire_gather(2, 0)
        fire_idx(3, 1)
        finish(1, 1, first=True)

        def pair_body(p, carry):
            t = 2 + 2 * p
            fire_gather(t + 1, 1)
            fire_idx(t + 2, 0)
            finish(t, 0, first=False)
            fire_gather(t + 2, 0)
            fire_idx(t + 3, 1)
            finish(t + 1, 1, first=False)
            return carry

        lax.fori_loop(0, (nchunks - 4) // 2, pair_body, 0)
        fire_gather(nchunks - 1, 1)
        finish(nchunks - 2, 0, first=False)
        finish(nchunks - 1, 1, first=False)
        wait_scatter(0)
        wait_scatter(1)

        plsc.subcore_barrier()

        def wblk(k, carry):
            roff = (s + k * NS) * RB
            pltpu.sync_copy(acc.at[pl.ds(roff, RB)], zb)
            pltpu.sync_copy(zb, aggr_hbm.at[pl.ds(roff, RB),
                                            pl.ds(coff, DH)])
            return carry

        lax.fori_loop(0, nblk, wblk, 0)

    return mega_kernel(P2, CE, row, col)


def kernel(X, E, emb_nodes, emb_edges, edge_index, edge_W, edge_b,
           node_W, node_b):
    N, D = X.shape
    En = E.shape[0]
    row = edge_index[0]
    col = edge_index[1]
    eb2 = edge_b.reshape(1, D)
    nb2 = node_b.reshape(1, D)
    W3 = lax.slice(edge_W, (2 * D, 0), (3 * D, D))

    P2 = _proj(X, edge_W, blk=2000)
    CE = _ce_pack(E, W3, eb2, blk=2000)
    E_new, aggr = _sc_mega(P2, CE, row, col, N, CB=40)
    X_new = _node_mlp(aggr, X, node_W, nb2, blk=2000)
    return X_new, E_new
